# baseline (speedup 1.0000x reference)
import functools

import jax
import jax.numpy as jnp
from jax import lax
from jax.experimental import pallas as pl
from jax.experimental.pallas import tpu as pltpu
from jax.experimental.pallas import tpu_sc as plsc

_B, _Q, _C1 = 32, 1000, 4
_T, _P = 100, 50
_PAIRS = _B * _T
_NQ = _B * _Q
_W = 128
_NW = _PAIRS // _W
_LANES = 16
_LABPAD = 128

def _sc_log(s):
    y = 2.0 * (s - 1.0) / (s + 1.0)
    for _ in range(3):
        y = y + s * jnp.exp(-y) - 1.0
    return y


def _sc_ce_body(logits_hbm, labels_hbm, out_hbm, lg_v, lab_v, acc_v):
    info = plsc.get_sparse_core_info()
    nc = info.num_cores
    wid = lax.axis_index("s") * nc + lax.axis_index("c")

    pltpu.sync_copy(logits_hbm.at[wid], lg_v)
    pltpu.sync_copy(labels_hbm.at[wid], lab_v)

    qbase = lax.broadcasted_iota(jnp.int32, (_LANES,), 0)
    starts = [16 * k for k in range(_Q // _LANES)] + [_Q - _LANES]
    acc = jnp.zeros((_LANES,), jnp.float32)
    for i, q0 in enumerate(starts):
        tail = i == len(starts) - 1
        l0 = lg_v[0, pl.ds(q0, _LANES)]
        l1 = lg_v[1, pl.ds(q0, _LANES)]
        l2 = lg_v[2, pl.ds(q0, _LANES)]
        l3 = lg_v[3, pl.ds(q0, _LANES)]
        m = jnp.maximum(jnp.maximum(l0, l1), jnp.maximum(l2, l3))
        s = (jnp.exp(l0 - m) + jnp.exp(l1 - m)
             + jnp.exp(l2 - m) + jnp.exp(l3 - m))
        lse = m + _sc_log(s)
        if q0 + _LANES <= _LABPAD:
            lab = lab_v[pl.ds(q0, _LANES)]
            matched = jnp.where(lab == 0, l0,
                                jnp.where(lab == 1, l1,
                                          jnp.where(lab == 2, l2, l3)))
        else:
            matched = l3
        contrib = lse - matched
        if tail:
            contrib = jnp.where(qbase + q0 >= 16 * (_Q // _LANES),
                                contrib, 0.0)
        acc = acc + contrib
    acc_v[...] = acc
    pltpu.sync_copy(acc_v, out_hbm.at[wid])


@functools.partial(
    pl.kernel,
    out_type=jax.ShapeDtypeStruct((_B, _LANES), jnp.float32),
    scratch_types=[
        pltpu.VMEM((_C1, _Q), jnp.float32),
        pltpu.VMEM((_LABPAD,), jnp.int32),
        pltpu.VMEM((_LANES,), jnp.float32),
    ],
    mesh=plsc.VectorSubcoreMesh(core_axis_name="c", subcore_axis_name="s"),
)
def _sc_ce(logits_hbm, labels_hbm, out_hbm, lg_v, lab_v, acc_v):
    _sc_ce_body(logits_hbm, labels_hbm, out_hbm, lg_v, lab_v, acc_v)


def _tc_kernel(s_ref, t_ref, out_ref):
    poly = 0.0
    for w in range(_NW):
        sl = slice(w * _W, (w + 1) * _W)
        sx = s_ref[0, :, sl]
        sy = s_ref[1, :, sl]
        tx = t_ref[0, :, sl]
        ty = t_ref[1, :, sl]
        macc = None
        acc1 = None
        for j in range(_P):
            txj = tx[j:j + 1]
            tyj = ty[j:j + 1]
            d = jnp.abs(sx - txj) + jnp.abs(sy - tyj)
            macc = d if macc is None else jnp.minimum(macc, d)
            cmin = jnp.min(d, axis=0, keepdims=True)
            acc1 = cmin if acc1 is None else acc1 + cmin
        per_t = acc1 + jnp.sum(macc, axis=0, keepdims=True)
        poly = poly + jnp.sum(per_t)
    poly = poly * (0.5 / (_PAIRS * _P))

    sdx = s_ref[0, _P - 1, :] - s_ref[0, 0, :]
    sdy = s_ref[1, _P - 1, :] - s_ref[1, 0, :]
    tdx = t_ref[0, _P - 1, :] - t_ref[0, 0, :]
    tdy = t_ref[1, _P - 1, :] - t_ref[1, 0, :]
    sn = jnp.sqrt(sdx * sdx + sdy * sdy) + 1e-6
    tn = jnp.sqrt(tdx * tdx + tdy * tdy) + 1e-6
    cos = (sdx * tdx + sdy * tdy) / (sn * tn)
    direc = jnp.sum(1.0 - cos) / _PAIRS

    idx = lax.broadcasted_iota(jnp.int32, (3,), 0)
    out_ref[...] = (jnp.where(idx == 1, poly, 0.0)
                    + jnp.where(idx == 2, direc, 0.0))


@jax.jit
def kernel(pred_logits, pred_polylines, tgt_labels, tgt_polylines):
    B, Q, C1 = pred_logits.shape
    T = tgt_labels.shape[1]
    P = pred_polylines.shape[2]

    logits_t = jnp.transpose(pred_logits, (0, 2, 1))
    labels_pad = jnp.concatenate(
        [tgt_labels.astype(jnp.int32),
         jnp.full((B, _LABPAD - T), C1 - 1, dtype=jnp.int32)], axis=1)
    ce_parts = _sc_ce(logits_t, labels_pad)
    ce = jnp.sum(ce_parts) / _NQ

    s_t = jnp.transpose(pred_polylines[:, :T], (3, 2, 0, 1)).reshape(2, P, B * T)
    t_t = jnp.transpose(tgt_polylines, (3, 2, 0, 1)).reshape(2, P, B * T)

    out = pl.pallas_call(
        _tc_kernel,
        out_shape=jax.ShapeDtypeStruct((3,), jnp.float32),
    )(s_t, t_t)
    idx = lax.iota(jnp.int32, 3)
    return out + jnp.where(idx == 0, ce, 0.0)
